# baseline (speedup 1.0000x reference)
import jax
import jax.numpy as jnp
from jax import lax
from jax.experimental import pallas as pl
from jax.experimental.pallas import tpu as pltpu
from jax.experimental.pallas import tpu_sc as plsc

BATCH = 16384
FACTORS = 64

_info = plsc.get_sparse_core_info()
_NC, _NS, _L = _info.num_cores, _info.num_subcores, _info.num_lanes
_NW = _NC * _NS
_BPW = BATCH // _NW
_GROUPS = _BPW // _L
_FCHUNK = 8


def _fm_body(user_hbm, item_hbm, eu_hbm, ei_hbm, ub_hbm, ib_hbm, b_hbm,
             out_hbm, uidx_v, iidx_v, ubuf_v, ibuf_v, ub_v, ib_v,
             bias_v, out_v, sem):
    wid = lax.axis_index("s") * _NC + lax.axis_index("c")
    base = wid * _BPW

    pltpu.sync_copy(user_hbm.at[pl.ds(base, _BPW)], uidx_v)
    pltpu.sync_copy(item_hbm.at[pl.ds(base, _BPW)], iidx_v)
    pltpu.sync_copy(b_hbm, bias_v)

    cp_ub = pltpu.async_copy(ub_hbm.at[uidx_v], ub_v, sem)
    cp_ib = pltpu.async_copy(ib_hbm.at[iidx_v], ib_v, sem)

    def fire(c, carry):
        for k in range(_FCHUNK):
            f = c * _FCHUNK + k
            pltpu.async_copy(eu_hbm.at[f].at[uidx_v], ubuf_v.at[f], sem)
            pltpu.async_copy(ei_hbm.at[f].at[iidx_v], ibuf_v.at[f], sem)
        return carry

    def drain(c, carry):
        for k in range(_FCHUNK):
            f = c * _FCHUNK + k
            pltpu.make_async_copy(eu_hbm.at[f].at[uidx_v], ubuf_v.at[f],
                                  sem).wait()
            pltpu.make_async_copy(ei_hbm.at[f].at[iidx_v], ibuf_v.at[f],
                                  sem).wait()
        return carry

    nrounds = FACTORS // _FCHUNK
    lax.fori_loop(0, nrounds, fire, 0)
    cp_ub.wait()
    cp_ib.wait()
    lax.fori_loop(0, nrounds, drain, 0)

    bias = bias_v[...]

    def group(g, carry):
        sl = pl.ds(g * _L, _L)
        acc = bias + ub_v[sl] + ib_v[sl]
        for f in range(FACTORS):
            acc = acc + ubuf_v[f, sl] * ibuf_v[f, sl]
        out_v[sl] = acc
        return carry

    lax.fori_loop(0, _GROUPS, group, 0)
    pltpu.sync_copy(out_v, out_hbm.at[pl.ds(base, _BPW)])


def kernel(user, item, embed_user, embed_item, u_bias, i_bias, bias_):
    mesh = plsc.VectorSubcoreMesh(core_axis_name="c", subcore_axis_name="s")
    fm = pl.kernel(
        _fm_body,
        out_type=jax.ShapeDtypeStruct((BATCH,), jnp.float32),
        mesh=mesh,
        compiler_params=pltpu.CompilerParams(
            needs_layout_passes=False, use_tc_tiling_on_sc=False),
        scratch_types=[
            pltpu.VMEM((_BPW,), jnp.int32),
            pltpu.VMEM((_BPW,), jnp.int32),
            pltpu.VMEM((FACTORS, _BPW), jnp.float32),
            pltpu.VMEM((FACTORS, _BPW), jnp.float32),
            pltpu.VMEM((_BPW,), jnp.float32),
            pltpu.VMEM((_BPW,), jnp.float32),
            pltpu.VMEM((_L,), jnp.float32),
            pltpu.VMEM((_BPW,), jnp.float32),
            pltpu.SemaphoreType.DMA,
        ],
    )
    return fm(user.astype(jnp.int32), item.astype(jnp.int32),
              embed_user.T, embed_item.T,
              u_bias.reshape(-1), i_bias.reshape(-1),
              jnp.broadcast_to(bias_, (_L,)))


# speedup vs baseline: 1.0010x; 1.0010x over previous
import jax
import jax.numpy as jnp
from jax import lax
from jax.experimental import pallas as pl
from jax.experimental.pallas import tpu as pltpu
from jax.experimental.pallas import tpu_sc as plsc

BATCH = 16384
FACTORS = 64
TABLE_N = 1000000

_info = plsc.get_sparse_core_info()
_NC, _NS, _L = _info.num_cores, _info.num_subcores, _info.num_lanes
_NW = _NC * _NS
_BPW = BATCH // _NW
_GROUPS = _BPW // _L
_FQ = 16
_NQ = FACTORS // _FQ
_QE = _FQ * _BPW


def _fm_body(user_hbm, item_hbm, eu_hbm, ei_hbm, ub_hbm, ib_hbm, b_hbm,
             out_hbm, uidx_v, iidx_v, uqidx0, uqidx1, iqidx0, iqidx1,
             ubuf0, ubuf1, ibuf0, ibuf1, ub_v, ib_v, bias_v, out_v,
             sem0, sem1, semb):
    wid = lax.axis_index("s") * _NC + lax.axis_index("c")
    base = wid * _BPW

    pltpu.sync_copy(user_hbm.at[pl.ds(base, _BPW)], uidx_v)
    pltpu.sync_copy(item_hbm.at[pl.ds(base, _BPW)], iidx_v)
    pltpu.sync_copy(b_hbm, bias_v)

    cp_ub = pltpu.async_copy(ub_hbm.at[uidx_v], ub_v, semb)
    cp_ib = pltpu.async_copy(ib_hbm.at[iidx_v], ib_v, semb)

    uqidx = (uqidx0, uqidx1)
    iqidx = (iqidx0, iqidx1)
    ubuf = (ubuf0, ubuf1)
    ibuf = (ibuf0, ibuf1)
    sems = (sem0, sem1)

    def build_idx(q, s):
        def body(g, carry):
            u16 = uidx_v[pl.ds(g * _L, _L)]
            i16 = iidx_v[pl.ds(g * _L, _L)]
            for k in range(_FQ):
                off = (q * _FQ + k) * TABLE_N
                uqidx[s][pl.ds(g * _L + k * _BPW, _L)] = u16 + off
                iqidx[s][pl.ds(g * _L + k * _BPW, _L)] = i16 + off
            return carry
        lax.fori_loop(0, _GROUPS, body, 0)

    def fire(s):
        pltpu.async_copy(eu_hbm.at[uqidx[s]], ubuf[s], sems[s])
        pltpu.async_copy(ei_hbm.at[iqidx[s]], ibuf[s], sems[s])

    def drain(s):
        pltpu.make_async_copy(eu_hbm.at[uqidx[s]], ubuf[s], sems[s]).wait()
        pltpu.make_async_copy(ei_hbm.at[iqidx[s]], ibuf[s], sems[s]).wait()

    def accumulate(s):
        def body(g, carry):
            sl = pl.ds(g * _L, _L)
            acc = out_v[sl]
            for k in range(_FQ):
                ksl = pl.ds(g * _L + k * _BPW, _L)
                acc = acc + ubuf[s][ksl] * ibuf[s][ksl]
            out_v[sl] = acc
            return carry
        lax.fori_loop(0, _GROUPS, body, 0)

    build_idx(0, 0)
    fire(0)

    cp_ub.wait()
    cp_ib.wait()
    bias = bias_v[...]

    def seed(g, carry):
        sl = pl.ds(g * _L, _L)
        out_v[sl] = bias + ub_v[sl] + ib_v[sl]
        return carry
    lax.fori_loop(0, _GROUPS, seed, 0)

    for q in range(_NQ):
        s = q % 2
        if q + 1 < _NQ:
            build_idx(q + 1, 1 - s)
            fire(1 - s)
        drain(s)
        accumulate(s)

    pltpu.sync_copy(out_v, out_hbm.at[pl.ds(base, _BPW)])


def kernel(user, item, embed_user, embed_item, u_bias, i_bias, bias_):
    mesh = plsc.VectorSubcoreMesh(core_axis_name="c", subcore_axis_name="s")
    fm = pl.kernel(
        _fm_body,
        out_type=jax.ShapeDtypeStruct((BATCH,), jnp.float32),
        mesh=mesh,
        compiler_params=pltpu.CompilerParams(
            needs_layout_passes=False, use_tc_tiling_on_sc=False),
        scratch_types=[
            pltpu.VMEM((_BPW,), jnp.int32),
            pltpu.VMEM((_BPW,), jnp.int32),
            pltpu.VMEM((_QE,), jnp.int32),
            pltpu.VMEM((_QE,), jnp.int32),
            pltpu.VMEM((_QE,), jnp.int32),
            pltpu.VMEM((_QE,), jnp.int32),
            pltpu.VMEM((_QE,), jnp.float32),
            pltpu.VMEM((_QE,), jnp.float32),
            pltpu.VMEM((_QE,), jnp.float32),
            pltpu.VMEM((_QE,), jnp.float32),
            pltpu.VMEM((_BPW,), jnp.float32),
            pltpu.VMEM((_BPW,), jnp.float32),
            pltpu.VMEM((_L,), jnp.float32),
            pltpu.VMEM((_BPW,), jnp.float32),
            pltpu.SemaphoreType.DMA,
            pltpu.SemaphoreType.DMA,
            pltpu.SemaphoreType.DMA,
        ],
    )
    return fm(user.astype(jnp.int32), item.astype(jnp.int32),
              embed_user.T.reshape(-1), embed_item.T.reshape(-1),
              u_bias.reshape(-1), i_bias.reshape(-1),
              jnp.broadcast_to(bias_, (_L,)))


# speedup vs baseline: 8.9419x; 8.9333x over previous
import jax
import jax.numpy as jnp
from jax import lax
from jax.experimental import pallas as pl
from jax.experimental.pallas import tpu as pltpu
from jax.experimental.pallas import tpu_sc as plsc

BATCH = 16384
FACTORS = 64
TABLE_N = 1000000
ROW2 = 2 * FACTORS

_info = plsc.get_sparse_core_info()
_NC, _NS, _L = _info.num_cores, _info.num_subcores, _info.num_lanes
_NW = _NC * _NS
_BPW = BATCH // _NW
_GROUPS = _BPW // _L
_CH = 128
_NCH = _BPW // _CH
_CG = _CH // _L


def _fm_body(user_hbm, item_hbm, eu_hbm, ei_hbm, ub_hbm, ib_hbm, b_hbm,
             out_hbm, uidx_v, iidx_v, ujidx0, ujidx1, ijidx0, ijidx1,
             ubuf0, ubuf1, ibuf0, ibuf1, ub_v, ib_v, bias_v, out_v,
             sem0, sem1, semb):
    wid = lax.axis_index("s") * _NC + lax.axis_index("c")
    base = wid * _BPW

    pltpu.sync_copy(user_hbm.at[pl.ds(base, _BPW)], uidx_v)
    pltpu.sync_copy(item_hbm.at[pl.ds(base, _BPW)], iidx_v)
    pltpu.sync_copy(b_hbm, bias_v)

    cp_ub = pltpu.async_copy(ub_hbm.at[uidx_v], ub_v, semb)
    cp_ib = pltpu.async_copy(ib_hbm.at[iidx_v], ib_v, semb)

    ujidx = (ujidx0, ujidx1)
    ijidx = (ijidx0, ijidx1)
    ubuf = (ubuf0, ubuf1)
    ibuf = (ibuf0, ibuf1)
    sems = (sem0, sem1)

    def build_and_fire(c, s):
        def body(g, carry):
            sl_src = pl.ds(c * _CH + g * _L, _L)
            sl_dst = pl.ds(g * _L, _L)
            ujidx[s][sl_dst] = uidx_v[sl_src] >> 1
            ijidx[s][sl_dst] = iidx_v[sl_src] >> 1
            return carry
        lax.fori_loop(0, _CG, body, 0)
        pltpu.async_copy(eu_hbm.at[ujidx[s]], ubuf[s], sems[s])
        pltpu.async_copy(ei_hbm.at[ijidx[s]], ibuf[s], sems[s])

    def drain(s):
        pltpu.make_async_copy(eu_hbm.at[ujidx[s]], ubuf[s], sems[s]).wait()
        pltpu.make_async_copy(ei_hbm.at[ijidx[s]], ibuf[s], sems[s]).wait()

    lanes = lax.iota(jnp.int32, _L)

    def extract(c, s):
        def body(g, carry):
            sl = pl.ds(c * _CH + g * _L, _L)
            row = g * _L + lanes
            uhalf = (uidx_v[sl] & 1) * FACTORS
            ihalf = (iidx_v[sl] & 1) * FACTORS
            acc = out_v[sl]
            for f in range(FACTORS):
                u = plsc.load_gather(ubuf[s], [row, uhalf + f])
                v = plsc.load_gather(ibuf[s], [row, ihalf + f])
                acc = acc + u * v
            out_v[sl] = acc
            return carry
        lax.fori_loop(0, _CG, body, 0)

    build_and_fire(0, 0)
    build_and_fire(1, 1)

    cp_ub.wait()
    cp_ib.wait()
    bias = bias_v[...]

    def seed(g, carry):
        sl = pl.ds(g * _L, _L)
        out_v[sl] = bias + ub_v[sl] + ib_v[sl]
        return carry
    lax.fori_loop(0, _GROUPS, seed, 0)

    for c in range(_NCH):
        s = c % 2
        drain(s)
        extract(c, s)
        if c + 2 < _NCH:
            build_and_fire(c + 2, s)

    pltpu.sync_copy(out_v, out_hbm.at[pl.ds(base, _BPW)])


def kernel(user, item, embed_user, embed_item, u_bias, i_bias, bias_):
    mesh = plsc.VectorSubcoreMesh(core_axis_name="c", subcore_axis_name="s")
    fm = pl.kernel(
        _fm_body,
        out_type=jax.ShapeDtypeStruct((BATCH,), jnp.float32),
        mesh=mesh,
        compiler_params=pltpu.CompilerParams(
            needs_layout_passes=False, use_tc_tiling_on_sc=False),
        scratch_types=[
            pltpu.VMEM((_BPW,), jnp.int32),
            pltpu.VMEM((_BPW,), jnp.int32),
            pltpu.VMEM((_CH,), jnp.int32),
            pltpu.VMEM((_CH,), jnp.int32),
            pltpu.VMEM((_CH,), jnp.int32),
            pltpu.VMEM((_CH,), jnp.int32),
            pltpu.VMEM((_CH, ROW2), jnp.float32),
            pltpu.VMEM((_CH, ROW2), jnp.float32),
            pltpu.VMEM((_CH, ROW2), jnp.float32),
            pltpu.VMEM((_CH, ROW2), jnp.float32),
            pltpu.VMEM((_BPW,), jnp.float32),
            pltpu.VMEM((_BPW,), jnp.float32),
            pltpu.VMEM((_L,), jnp.float32),
            pltpu.VMEM((_BPW,), jnp.float32),
            pltpu.SemaphoreType.DMA,
            pltpu.SemaphoreType.DMA,
            pltpu.SemaphoreType.DMA,
        ],
    )
    return fm(user.astype(jnp.int32), item.astype(jnp.int32),
              embed_user.reshape(TABLE_N // 2, ROW2),
              embed_item.reshape(TABLE_N // 2, ROW2),
              u_bias.reshape(-1), i_bias.reshape(-1),
              jnp.broadcast_to(bias_, (_L,)))
